# baseline (speedup 1.0000x reference)
import functools

import jax
import jax.numpy as jnp
from jax import lax
from jax.experimental import pallas as pl
from jax.experimental.pallas import tpu as pltpu
from jax.experimental.pallas import tpu_sc as plsc

_MAX_VAL = 10000.0
_KS = (1, 5, 10, 20, 50, 100)

_B, _V, _L = 1024, 100000, 200
_LP = 208
_PAD_COL = _V

_NC, _NS = 2, 16
_NW = _NC * _NS
_RPW = _B // _NW
_RB = _V + 16

_BB = 32
_BB3 = 64


def _count_half(scores_ref, labels_ref, out_ref):
  s = scores_ref[...]
  preds = []
  for r in range(_BB):
    lab = labels_ref[r, 0]
    t0 = pl.multiple_of((lab // 128) * 128, 128)
    tile = scores_ref[pl.ds(r, 1), pl.ds(t0, 128)]
    lane = lax.broadcasted_iota(jnp.int32, (1, 128), 1) == (lab % 128)
    preds.append(jnp.sum(jnp.where(lane, tile, 0.0), axis=1, keepdims=True))
  pred = jnp.concatenate(preds, axis=0)
  c1 = jnp.sum((pred < s).astype(jnp.float32), axis=1, keepdims=True)
  c2 = jnp.sum((s > -_MAX_VAL).astype(jnp.float32), axis=1, keepdims=True)
  out_ref[...] = jnp.concatenate(
      [c1, c2, pred, jnp.zeros_like(pred)], axis=1)


def _k1_body(sa_ref, sb_ref, la_ref, lb_ref, outa_ref, outb_ref):
  _count_half(sa_ref, la_ref, outa_ref)
  _count_half(sb_ref, lb_ref, outb_ref)


def _k2_body(sidx_hbm, vals_hbm, out_hbm, arena_v, idx_v, val_v, out_v):
  wid = lax.axis_index("s") * _NC + lax.axis_index("c")
  base = wid * (_RPW * _LP)
  pltpu.sync_copy(sidx_hbm.at[pl.ds(base, _RPW * _LP)], idx_v)
  pltpu.sync_copy(vals_hbm.at[pl.ds(base, _RPW * _LP)], val_v)

  @pl.loop(0, _RPW)
  def _(i):
    for k in range(13):
      cidx = idx_v[pl.ds(i * _LP + k * 16, 16)]
      jid = lax.iota(jnp.int32, 16) + (k * 16)
      plsc.store_scatter(arena_v, [cidx], jid)
    for k in range(13):
      cidx = idx_v[pl.ds(i * _LP + k * 16, 16)]
      jid = lax.iota(jnp.int32, 16) + (k * 16)
      mark = plsc.load_gather(arena_v, [cidx])
      keep = (mark == jid) & (jid < _L)
      out_v[pl.ds(i * _LP + k * 16, 16)] = jnp.where(
          keep, val_v[pl.ds(i * _LP + k * 16, 16)], jnp.float32(-_MAX_VAL))

  pltpu.sync_copy(out_v, out_hbm.at[pl.ds(base, _RPW * _LP)])


@functools.cache
def _make_k2():
  return pl.kernel(
      _k2_body,
      out_type=jax.ShapeDtypeStruct((_B * _LP,), jnp.float32),
      mesh=plsc.VectorSubcoreMesh(core_axis_name="c", subcore_axis_name="s",
                                  num_cores=_NC, num_subcores=_NS),
      compiler_params=pltpu.CompilerParams(needs_layout_passes=False),
      scratch_types=[
          pltpu.VMEM((_RB,), jnp.int32),
          pltpu.VMEM((_RPW * _LP,), jnp.int32),
          pltpu.VMEM((_RPW * _LP,), jnp.float32),
          pltpu.VMEM((_RPW * _LP,), jnp.float32),
      ],
  )


def _k3_body(cnt_ref, svp_ref, out_ref):
  step = pl.program_id(0)
  cnt = cnt_ref[...]
  svp = svp_ref[...]
  c1 = cnt[:, 0:1]
  c2 = cnt[:, 1:2]
  pred = cnt[:, 2:3]
  ltm = (pred < -_MAX_VAL).astype(jnp.float32)
  corr1 = (jnp.sum((pred < svp).astype(jnp.float32), axis=1, keepdims=True)
           - float(_LP) * ltm)
  corr2 = jnp.sum((svp > -_MAX_VAL).astype(jnp.float32),
                  axis=1, keepdims=True)
  rank = c1 - corr1
  valid = c2 - corr2

  dcg = 1.0 / jnp.log2(rank + 2.0)
  cols = []
  for k in _KS:
    ind = (rank < float(k)).astype(jnp.float32)
    cols.append(dcg * ind)
    cols.append(ind)
  cols.append(1.0 / (rank + 1.0))
  cols.append(1.0 - rank / valid)
  cols.append(jnp.zeros_like(rank))
  cols.append(jnp.zeros_like(rank))
  part = jnp.sum(jnp.concatenate(cols, axis=1), axis=0, keepdims=True)

  @pl.when(step == 0)
  def _():
    out_ref[...] = jnp.zeros_like(out_ref)

  out_ref[...] += part * (1.0 / _B)


def kernel(scores, labels, seqs):
  pad = jnp.full((_B, _LP - _L), _PAD_COL, dtype=jnp.int32)
  sidx = jnp.concatenate([seqs, pad], axis=1)
  g = jnp.take_along_axis(scores, jnp.minimum(sidx, _V - 1), axis=1)

  svp = _make_k2()(sidx.reshape(-1), g.reshape(-1)).reshape(_B, _LP)

  nhalf = _B // (2 * _BB)
  cnta, cntb = pl.pallas_call(
      _k1_body,
      grid=(nhalf,),
      in_specs=[
          pl.BlockSpec((_BB, _V), lambda i: (i, 0)),
          pl.BlockSpec((_BB, _V), lambda i: (i + nhalf, 0)),
          pl.BlockSpec((_BB, 1), lambda i: (i, 0),
                       memory_space=pltpu.SMEM),
          pl.BlockSpec((_BB, 1), lambda i: (i + nhalf, 0),
                       memory_space=pltpu.SMEM),
      ],
      out_specs=[
          pl.BlockSpec((_BB, 4), lambda i: (i, 0)),
          pl.BlockSpec((_BB, 4), lambda i: (i, 0)),
      ],
      out_shape=[
          jax.ShapeDtypeStruct((_B // 2, 4), jnp.float32),
          jax.ShapeDtypeStruct((_B // 2, 4), jnp.float32),
      ],
  )(scores, scores, labels, labels)
  cnt = jnp.concatenate([cnta, cntb], axis=0)

  out = pl.pallas_call(
      _k3_body,
      grid=(_B // _BB3,),
      in_specs=[
          pl.BlockSpec((_BB3, 4), lambda i: (i, 0)),
          pl.BlockSpec((_BB3, _LP), lambda i: (i, 0)),
      ],
      out_specs=pl.BlockSpec((1, 16), lambda i: (0, 0)),
      out_shape=jax.ShapeDtypeStruct((1, 16), jnp.float32),
  )(cnt, svp)
  return out[0, :15]


# speedup vs baseline: 1.0082x; 1.0082x over previous
import functools

import jax
import jax.numpy as jnp
from jax import lax
from jax.experimental import pallas as pl
from jax.experimental.pallas import tpu as pltpu
from jax.experimental.pallas import tpu_sc as plsc

_MAX_VAL = 10000.0
_KS = (1, 5, 10, 20, 50, 100)

_B, _V, _L = 1024, 100000, 200
_LP = 208
_PAD_COL = _V

_NC, _NS = 2, 16
_NW = _NC * _NS
_RPW = _B // _NW
_RB = _V + 16

_BB = 64
_BB3 = 64


def _k1_body(scores_ref, labels_ref, out_ref):
  s = scores_ref[...]
  preds = []
  for r in range(_BB):
    lab = labels_ref[r, 0]
    t0 = pl.multiple_of((lab // 128) * 128, 128)
    tile = scores_ref[pl.ds(r, 1), pl.ds(t0, 128)]
    lane = lax.broadcasted_iota(jnp.int32, (1, 128), 1) == (lab % 128)
    preds.append(jnp.sum(jnp.where(lane, tile, 0.0), axis=1, keepdims=True))
  pred = jnp.concatenate(preds, axis=0)
  c1 = jnp.sum((pred < s).astype(jnp.float32), axis=1, keepdims=True)
  c2 = jnp.sum((s > -_MAX_VAL).astype(jnp.float32), axis=1, keepdims=True)
  out_ref[...] = jnp.concatenate(
      [c1, c2, pred, jnp.zeros_like(pred)], axis=1)


def _k2_body(sidx_hbm, vals_hbm, out_hbm, arena_v, idx_v, val_v, out_v):
  wid = lax.axis_index("s") * _NC + lax.axis_index("c")
  base = wid * (_RPW * _LP)
  pltpu.sync_copy(sidx_hbm.at[pl.ds(base, _RPW * _LP)], idx_v)
  pltpu.sync_copy(vals_hbm.at[pl.ds(base, _RPW * _LP)], val_v)

  @pl.loop(0, _RPW)
  def _(i):
    for k in range(13):
      cidx = idx_v[pl.ds(i * _LP + k * 16, 16)]
      jid = lax.iota(jnp.int32, 16) + (k * 16)
      plsc.store_scatter(arena_v, [cidx], jid)
    for k in range(13):
      cidx = idx_v[pl.ds(i * _LP + k * 16, 16)]
      jid = lax.iota(jnp.int32, 16) + (k * 16)
      mark = plsc.load_gather(arena_v, [cidx])
      keep = (mark == jid) & (jid < _L)
      out_v[pl.ds(i * _LP + k * 16, 16)] = jnp.where(
          keep, val_v[pl.ds(i * _LP + k * 16, 16)], jnp.float32(-_MAX_VAL))

  pltpu.sync_copy(out_v, out_hbm.at[pl.ds(base, _RPW * _LP)])


@functools.cache
def _make_k2():
  return pl.kernel(
      _k2_body,
      out_type=jax.ShapeDtypeStruct((_B * _LP,), jnp.float32),
      mesh=plsc.VectorSubcoreMesh(core_axis_name="c", subcore_axis_name="s",
                                  num_cores=_NC, num_subcores=_NS),
      compiler_params=pltpu.CompilerParams(needs_layout_passes=False),
      scratch_types=[
          pltpu.VMEM((_RB,), jnp.int32),
          pltpu.VMEM((_RPW * _LP,), jnp.int32),
          pltpu.VMEM((_RPW * _LP,), jnp.float32),
          pltpu.VMEM((_RPW * _LP,), jnp.float32),
      ],
  )


def _k3_body(cnt_ref, svp_ref, out_ref):
  step = pl.program_id(0)
  cnt = cnt_ref[...]
  svp = svp_ref[...]
  c1 = cnt[:, 0:1]
  c2 = cnt[:, 1:2]
  pred = cnt[:, 2:3]
  ltm = (pred < -_MAX_VAL).astype(jnp.float32)
  corr1 = (jnp.sum((pred < svp).astype(jnp.float32), axis=1, keepdims=True)
           - float(_LP) * ltm)
  corr2 = jnp.sum((svp > -_MAX_VAL).astype(jnp.float32),
                  axis=1, keepdims=True)
  rank = c1 - corr1
  valid = c2 - corr2

  dcg = 1.0 / jnp.log2(rank + 2.0)
  cols = []
  for k in _KS:
    ind = (rank < float(k)).astype(jnp.float32)
    cols.append(dcg * ind)
    cols.append(ind)
  cols.append(1.0 / (rank + 1.0))
  cols.append(1.0 - rank / valid)
  cols.append(jnp.zeros_like(rank))
  cols.append(jnp.zeros_like(rank))
  part = jnp.sum(jnp.concatenate(cols, axis=1), axis=0, keepdims=True)

  @pl.when(step == 0)
  def _():
    out_ref[...] = jnp.zeros_like(out_ref)

  out_ref[...] += part * (1.0 / _B)


def kernel(scores, labels, seqs):
  pad = jnp.full((_B, _LP - _L), _PAD_COL, dtype=jnp.int32)
  sidx = jnp.concatenate([seqs, pad], axis=1)
  g = jnp.take_along_axis(scores, jnp.minimum(sidx, _V - 1), axis=1)

  svp = _make_k2()(sidx.reshape(-1), g.reshape(-1)).reshape(_B, _LP)

  cnt = pl.pallas_call(
      _k1_body,
      grid=(_B // _BB,),
      in_specs=[
          pl.BlockSpec((_BB, _V), lambda i: (i, 0)),
          pl.BlockSpec((_BB, 1), lambda i: (i, 0),
                       memory_space=pltpu.SMEM),
      ],
      out_specs=pl.BlockSpec((_BB, 4), lambda i: (i, 0)),
      out_shape=jax.ShapeDtypeStruct((_B, 4), jnp.float32),
  )(scores, labels)

  out = pl.pallas_call(
      _k3_body,
      grid=(_B // _BB3,),
      in_specs=[
          pl.BlockSpec((_BB3, 4), lambda i: (i, 0)),
          pl.BlockSpec((_BB3, _LP), lambda i: (i, 0)),
      ],
      out_specs=pl.BlockSpec((1, 16), lambda i: (0, 0)),
      out_shape=jax.ShapeDtypeStruct((1, 16), jnp.float32),
  )(cnt, svp)
  return out[0, :15]
